# Initial kernel scaffold; baseline (speedup 1.0000x reference)
import jax
import jax.numpy as jnp
from jax.experimental import pallas as pl


def kernel(token_ids, bitwidths, signed_flags, phy_types, target_mask, dependency_mask, fanout_mask, token_table, bw_W1, bw_b1, bw_W2, bw_b2, sg_W1, sg_b1, sg_W2, sg_b2, type_table, tgt_W, tgt_b, dep_W, dep_b, fan_W, fan_b, pos_table, ln_g, ln_b):
    raise NotImplementedError("write your pallas kernel here")


# speedup vs baseline: 2.4875x; 2.4875x over previous
import functools

import jax
import jax.numpy as jnp
from jax import lax
from jax.experimental import pallas as pl
from jax.experimental.pallas import tpu as pltpu
from jax.experimental.pallas import tpu_sc as plsc


_CHUNK = 128


def _make_sc_gather(V, D, BL):
    info = plsc.get_sparse_core_info()
    NC, NS = info.num_cores, info.num_subcores
    NW = NC * NS
    assert BL % (NW * _CHUNK) == 0
    per_w = BL // NW
    n_chunks = per_w // _CHUNK

    mesh = plsc.VectorSubcoreMesh(core_axis_name="c", subcore_axis_name="s")

    @functools.partial(
        pl.kernel,
        mesh=mesh,
        out_type=jax.ShapeDtypeStruct((BL, D), jnp.float32),
        scratch_types=[
            pltpu.VMEM((n_chunks, _CHUNK), jnp.int32),
            pltpu.VMEM((_CHUNK, D), jnp.float32),
            pltpu.SemaphoreType.DMA,
        ],
    )
    def gather_kernel(table_hbm, ids_hbm, out_hbm, idx_v, rows_v, gsem):
        wid = lax.axis_index("s") * NC + lax.axis_index("c")
        pltpu.sync_copy(ids_hbm.at[wid], idx_v)
        base = wid * per_w

        def step(j, carry):
            pltpu.async_copy(table_hbm.at[idx_v.at[j]], rows_v, gsem).wait()
            pltpu.sync_copy(rows_v, out_hbm.at[pl.ds(base + j * _CHUNK, _CHUNK), :])
            return carry

        lax.fori_loop(0, n_chunks, step, 0)

    return gather_kernel


def _dense_body(g_ref, bw_ref, sg_ref, phy_ref, tgt_ref, dep_ref, fan_ref,
                w1b_ref, b1b_ref, W2b_ref, b2b_ref,
                w1s_ref, b1s_ref, W2s_ref, b2s_ref,
                tt_ref, tw_ref, tb_ref, dw_ref, db_ref, fw_ref, fb_ref,
                pos_ref, lng_ref, lnb_ref, out_ref):
    x = g_ref[...]
    BB, L, D = x.shape

    bw = bw_ref[...]
    h = jnp.maximum(bw * w1b_ref[...][None, None, :]
                    + b1b_ref[...][None, None, :], 0.0)
    x = x + jnp.dot(h.reshape(BB * L, -1), W2b_ref[...],
                    preferred_element_type=jnp.float32).reshape(BB, L, D)

    sg = sg_ref[...]
    h = jnp.maximum(sg * w1s_ref[...][None, None, :]
                    + b1s_ref[...][None, None, :], 0.0)
    x = x + jnp.dot(h.reshape(BB * L, -1), W2s_ref[...],
                    preferred_element_type=jnp.float32).reshape(BB, L, D)

    phy = phy_ref[...]
    tadd = jnp.zeros_like(x)
    for k in range(tt_ref.shape[0]):
        tadd = tadd + jnp.where(phy == k, 1.0, 0.0) * tt_ref[k][None, None, :]
    x = x + tadd

    bias = (b2b_ref[...] + b2s_ref[...] + tb_ref[...] + db_ref[...] + fb_ref[...])
    x = x + (tgt_ref[...] * tw_ref[...][None, None, :]
             + dep_ref[...] * dw_ref[...][None, None, :]
             + fan_ref[...] * fw_ref[...][None, None, :]
             + bias[None, None, :] + pos_ref[...][None, :, :])

    mu = jnp.mean(x, axis=-1, keepdims=True)
    var = jnp.mean((x - mu) ** 2, axis=-1, keepdims=True)
    out_ref[...] = ((x - mu) * lax.rsqrt(var + 1e-5)
                    * lng_ref[...][None, None, :] + lnb_ref[...][None, None, :])


def _dense_call(g3, bw, sg, phy, tgt, dep, fan,
                w1b, b1b, W2b, b2b, w1s, b1s, W2s, b2s,
                tt, tw, tb, dw, db, fw, fb, pos, lng, lnb, BB=8):
    B, L, D = g3.shape

    def bl3(i):
        return (i, 0, 0)

    def rep2(i):
        return (0, 0)

    def rep1(i):
        return (0,)

    tok3 = pl.BlockSpec((BB, L, 1), bl3)
    vecD = pl.BlockSpec(lnb.shape, rep1)

    return pl.pallas_call(
        _dense_body,
        grid=(B // BB,),
        in_specs=[
            pl.BlockSpec((BB, L, D), bl3),
            tok3, tok3, tok3, tok3, tok3, tok3,
            pl.BlockSpec(w1b.shape, rep1), pl.BlockSpec(b1b.shape, rep1),
            pl.BlockSpec(W2b.shape, rep2), vecD,
            pl.BlockSpec(w1s.shape, rep1), pl.BlockSpec(b1s.shape, rep1),
            pl.BlockSpec(W2s.shape, rep2), vecD,
            pl.BlockSpec(tt.shape, rep2),
            vecD, vecD, vecD, vecD, vecD, vecD,
            pl.BlockSpec(pos.shape, rep2),
            vecD, vecD,
        ],
        out_specs=pl.BlockSpec((BB, L, D), bl3),
        out_shape=jax.ShapeDtypeStruct((B, L, D), jnp.float32),
    )(g3, bw, sg, phy, tgt, dep, fan,
      w1b, b1b, W2b, b2b, w1s, b1s, W2s, b2s,
      tt, tw, tb, dw, db, fw, fb, pos, lng, lnb)


def kernel(token_ids, bitwidths, signed_flags, phy_types, target_mask,
           dependency_mask, fanout_mask, token_table, bw_W1, bw_b1, bw_W2,
           bw_b2, sg_W1, sg_b1, sg_W2, sg_b2, type_table, tgt_W, tgt_b,
           dep_W, dep_b, fan_W, fan_b, pos_table, ln_g, ln_b):
    B, L = token_ids.shape
    V, D = token_table.shape
    BL = B * L

    info = plsc.get_sparse_core_info()
    nw = info.num_cores * info.num_subcores
    ids3d = token_ids.astype(jnp.int32).reshape(nw, BL // (nw * _CHUNK), _CHUNK)
    gathered = _make_sc_gather(V, D, BL)(token_table, ids3d)
    g3 = gathered.reshape(B, L, D)

    def unsq(a):
        return a.reshape(B, L, 1)

    return _dense_call(
        g3, unsq(bitwidths), unsq(signed_flags),
        unsq(phy_types.astype(jnp.int32)),
        unsq(target_mask), unsq(dependency_mask), unsq(fanout_mask),
        bw_W1.reshape(-1), bw_b1, bw_W2, bw_b2,
        sg_W1.reshape(-1), sg_b1, sg_W2, sg_b2,
        type_table, tgt_W.reshape(-1), tgt_b, dep_W.reshape(-1), dep_b,
        fan_W.reshape(-1), fan_b, pos_table[:L], ln_g, ln_b)


# speedup vs baseline: 6.9707x; 2.8022x over previous
import functools

import jax
import jax.numpy as jnp
from jax import lax
from jax.experimental import pallas as pl
from jax.experimental.pallas import tpu as pltpu
from jax.experimental.pallas import tpu_sc as plsc


_CHUNK = 128


def _make_sc_gather(V, D, BL):
    info = plsc.get_sparse_core_info()
    NC, NS = info.num_cores, info.num_subcores
    NW = NC * NS
    assert BL % (NW * _CHUNK) == 0
    per_w = BL // NW
    n_chunks = per_w // _CHUNK

    mesh = plsc.VectorSubcoreMesh(core_axis_name="c", subcore_axis_name="s")

    @functools.partial(
        pl.kernel,
        mesh=mesh,
        out_type=jax.ShapeDtypeStruct((BL, D), jnp.float32),
        scratch_types=[
            pltpu.VMEM((n_chunks, _CHUNK), jnp.int32),
            pltpu.VMEM((_CHUNK, D), jnp.float32),
            pltpu.SemaphoreType.DMA,
        ],
    )
    def gather_kernel(table_hbm, ids_hbm, out_hbm, idx_v, rows_v, gsem):
        wid = lax.axis_index("s") * NC + lax.axis_index("c")
        pltpu.sync_copy(ids_hbm.at[wid], idx_v)
        base = wid * per_w

        def step(j, carry):
            pltpu.async_copy(table_hbm.at[idx_v.at[j]], rows_v, gsem).wait()
            pltpu.sync_copy(rows_v, out_hbm.at[pl.ds(base + j * _CHUNK, _CHUNK), :])
            return carry

        lax.fori_loop(0, n_chunks, step, 0)

    return gather_kernel


_H1 = 64
_H2 = 32
_NT = 6


def _dense_body(g_ref, mk_ref, w1_ref, b1_ref, Wc_ref, pos_ref,
                lng_ref, lnb_ref, out_ref):
    x = g_ref[...]
    R, D = x.shape
    H = _H1 + _H2

    mk = mk_ref[...]
    bw = mk[0:1, :]
    sg = mk[1:2, :]
    rank1 = mk[2:5, :]
    phyf = mk[5:6, :]

    drive = jnp.where(lax.broadcasted_iota(jnp.int32, (H, 1), 0) < _H1, bw, sg)
    hid = jnp.maximum(w1_ref[...] * drive + b1_ref[...], 0.0)

    oh = (phyf == lax.broadcasted_iota(jnp.int32, (_NT, 1), 0).astype(jnp.float32)
          ).astype(jnp.float32)

    ones = jnp.full((1, R), 1.0, jnp.float32)
    F = jnp.concatenate([hid, oh, rank1, ones], axis=0)

    x = x + lax.dot_general(F, Wc_ref[...], (((0,), (0,)), ((), ())),
                            preferred_element_type=jnp.float32)
    x = x + pos_ref[...]

    onesD = jnp.full((D, 1), 1.0 / D, jnp.float32)
    mu = jnp.dot(x, onesD, preferred_element_type=jnp.float32)
    ms = jnp.dot(x * x, onesD, preferred_element_type=jnp.float32)
    var = ms - mu * mu
    s = lax.rsqrt(var + 1e-5) * lng_ref[...][None, :]
    out_ref[...] = (x - mu) * s + lnb_ref[...][None, :]


def _dense_call(g2, mk, w1cat, b1cat, Wcat, pos_tiled, lng, lnb, R=3200):
    BL, D = g2.shape

    def bl2(i):
        return (i, 0)

    def rep2(i):
        return (0, 0)

    def rep1(i):
        return (0,)

    vecD = pl.BlockSpec(lnb.shape, rep1)

    return pl.pallas_call(
        _dense_body,
        grid=(BL // R,),
        in_specs=[
            pl.BlockSpec((R, D), bl2),
            pl.BlockSpec((mk.shape[0], R), lambda i: (0, i)),
            pl.BlockSpec(w1cat.shape, rep2), pl.BlockSpec(b1cat.shape, rep2),
            pl.BlockSpec(Wcat.shape, rep2),
            pl.BlockSpec((R, D), rep2),
            vecD, vecD,
        ],
        out_specs=pl.BlockSpec((R, D), bl2),
        out_shape=jax.ShapeDtypeStruct((BL, D), jnp.float32),
        compiler_params=pltpu.CompilerParams(
            dimension_semantics=("arbitrary",)),
    )(g2, mk, w1cat, b1cat, Wcat, pos_tiled, lng, lnb)


def kernel(token_ids, bitwidths, signed_flags, phy_types, target_mask,
           dependency_mask, fanout_mask, token_table, bw_W1, bw_b1, bw_W2,
           bw_b2, sg_W1, sg_b1, sg_W2, sg_b2, type_table, tgt_W, tgt_b,
           dep_W, dep_b, fan_W, fan_b, pos_table, ln_g, ln_b):
    B, L = token_ids.shape
    V, D = token_table.shape
    BL = B * L

    info = plsc.get_sparse_core_info()
    nw = info.num_cores * info.num_subcores
    ids3d = token_ids.astype(jnp.int32).reshape(nw, BL // (nw * _CHUNK), _CHUNK)
    gathered = _make_sc_gather(V, D, BL)(token_table, ids3d)

    R = 3200
    assert R % L == 0 and BL % R == 0 and R % 128 == 0
    pos_tiled = jnp.tile(pos_table[:L], (R // L, 1))

    def row(a):
        return a.reshape(1, BL).astype(jnp.float32)

    mk = jnp.concatenate([row(bitwidths), row(signed_flags), row(target_mask),
                          row(dependency_mask), row(fanout_mask),
                          row(phy_types)], axis=0)

    w1cat = jnp.concatenate([bw_W1.reshape(-1), sg_W1.reshape(-1)]).reshape(-1, 1)
    b1cat = jnp.concatenate([bw_b1, sg_b1]).reshape(-1, 1)
    bias_total = bw_b2 + sg_b2 + tgt_b + dep_b + fan_b
    Wcat = jnp.concatenate([bw_W2, sg_W2, type_table, tgt_W, dep_W,
                            fan_W, bias_total[None, :]], axis=0)

    out2 = _dense_call(gathered, mk, w1cat, b1cat, Wcat, pos_tiled,
                       ln_g, ln_b, R=R)
    return out2.reshape(B, L, D)


# speedup vs baseline: 7.5155x; 1.0782x over previous
import functools

import jax
import jax.numpy as jnp
from jax import lax
from jax.experimental import pallas as pl
from jax.experimental.pallas import tpu as pltpu
from jax.experimental.pallas import tpu_sc as plsc


_CHUNK = 128


def _make_sc_gather(V, D, BL):
    info = plsc.get_sparse_core_info()
    NC, NS = info.num_cores, info.num_subcores
    NW = NC * NS
    assert BL % (NW * _CHUNK) == 0
    per_w = BL // NW
    n_chunks = per_w // _CHUNK

    mesh = plsc.VectorSubcoreMesh(core_axis_name="c", subcore_axis_name="s")

    @functools.partial(
        pl.kernel,
        mesh=mesh,
        out_type=jax.ShapeDtypeStruct((BL, D), jnp.float32),
        scratch_types=[
            pltpu.VMEM((n_chunks, _CHUNK), jnp.int32),
            pltpu.VMEM((_CHUNK, D), jnp.float32),
            pltpu.VMEM((_CHUNK, D), jnp.float32),
            pltpu.SemaphoreType.DMA,
            pltpu.SemaphoreType.DMA,
        ],
    )
    def gather_kernel(table_hbm, ids_hbm, out_hbm, idx_v, buf_a, buf_b, sem_a, sem_b):
        wid = lax.axis_index("s") * NC + lax.axis_index("c")
        pltpu.sync_copy(ids_hbm.at[wid], idx_v)
        base = wid * per_w

        def gstart(j, buf, sem):
            pltpu.async_copy(table_hbm.at[idx_v.at[j]], buf, sem)

        def gwait(j, buf, sem):
            pltpu.make_async_copy(table_hbm.at[idx_v.at[j]], buf, sem).wait()

        def store(j, buf):
            pltpu.sync_copy(buf, out_hbm.at[pl.ds(base + j * _CHUNK, _CHUNK), :])

        gstart(0, buf_a, sem_a)

        def step(i, carry):
            j0 = 2 * i
            j1 = j0 + 1
            gwait(j0, buf_a, sem_a)
            gstart(j1, buf_b, sem_b)
            store(j0, buf_a)
            gwait(j1, buf_b, sem_b)

            @pl.when(i < n_chunks // 2 - 1)
            def _():
                gstart(j0 + 2, buf_a, sem_a)

            store(j1, buf_b)
            return carry

        lax.fori_loop(0, n_chunks // 2, step, 0)

    return gather_kernel


_H1 = 64
_H2 = 32
_NT = 6


def _dense_body(g_ref, mk_ref, w1_ref, b1_ref, Wc_ref, pos_ref,
                lng_ref, lnb_ref, out_ref):
    x = g_ref[...]
    R, D = x.shape
    H = _H1 + _H2

    mk = mk_ref[...]
    bw = mk[0:1, :]
    sg = mk[1:2, :]
    rank1 = mk[2:5, :]
    phyf = mk[5:6, :]

    drive = jnp.where(lax.broadcasted_iota(jnp.int32, (H, 1), 0) < _H1, bw, sg)
    hid = jnp.maximum(w1_ref[...] * drive + b1_ref[...], 0.0)

    oh = (phyf == lax.broadcasted_iota(jnp.int32, (_NT, 1), 0).astype(jnp.float32)
          ).astype(jnp.float32)

    ones = jnp.full((1, R), 1.0, jnp.float32)
    F = jnp.concatenate([hid, oh, rank1, ones], axis=0)

    x = x + lax.dot_general(F, Wc_ref[...], (((0,), (0,)), ((), ())),
                            preferred_element_type=jnp.float32)
    x = x + pos_ref[...]

    onesD = jnp.full((D, 1), 1.0 / D, jnp.float32)
    mu = jnp.dot(x, onesD, preferred_element_type=jnp.float32)
    ms = jnp.dot(x * x, onesD, preferred_element_type=jnp.float32)
    var = ms - mu * mu
    s = lax.rsqrt(var + 1e-5) * lng_ref[...][None, :]
    out_ref[...] = (x - mu) * s + lnb_ref[...][None, :]


def _dense_call(g2, mk, w1cat, b1cat, Wcat, pos_tiled, lng, lnb, R=3200):
    BL, D = g2.shape

    def bl2(i):
        return (i, 0)

    def rep2(i):
        return (0, 0)

    def rep1(i):
        return (0,)

    vecD = pl.BlockSpec(lnb.shape, rep1)

    return pl.pallas_call(
        _dense_body,
        grid=(BL // R,),
        in_specs=[
            pl.BlockSpec((R, D), bl2),
            pl.BlockSpec((mk.shape[0], R), lambda i: (0, i)),
            pl.BlockSpec(w1cat.shape, rep2), pl.BlockSpec(b1cat.shape, rep2),
            pl.BlockSpec(Wcat.shape, rep2),
            pl.BlockSpec((R, D), rep2),
            vecD, vecD,
        ],
        out_specs=pl.BlockSpec((R, D), bl2),
        out_shape=jax.ShapeDtypeStruct((BL, D), jnp.float32),
        compiler_params=pltpu.CompilerParams(
            dimension_semantics=("arbitrary",)),
    )(g2, mk, w1cat, b1cat, Wcat, pos_tiled, lng, lnb)


def kernel(token_ids, bitwidths, signed_flags, phy_types, target_mask,
           dependency_mask, fanout_mask, token_table, bw_W1, bw_b1, bw_W2,
           bw_b2, sg_W1, sg_b1, sg_W2, sg_b2, type_table, tgt_W, tgt_b,
           dep_W, dep_b, fan_W, fan_b, pos_table, ln_g, ln_b):
    B, L = token_ids.shape
    V, D = token_table.shape
    BL = B * L

    info = plsc.get_sparse_core_info()
    nw = info.num_cores * info.num_subcores
    ids3d = token_ids.astype(jnp.int32).reshape(nw, BL // (nw * _CHUNK), _CHUNK)
    gathered = _make_sc_gather(V, D, BL)(token_table, ids3d)

    R = 3200
    assert R % L == 0 and BL % R == 0 and R % 128 == 0
    pos_tiled = jnp.tile(pos_table[:L], (R // L, 1))

    def row(a):
        return a.reshape(1, BL).astype(jnp.float32)

    mk = jnp.concatenate([row(bitwidths), row(signed_flags), row(target_mask),
                          row(dependency_mask), row(fanout_mask),
                          row(phy_types)], axis=0)

    w1cat = jnp.concatenate([bw_W1.reshape(-1), sg_W1.reshape(-1)]).reshape(-1, 1)
    b1cat = jnp.concatenate([bw_b1, sg_b1]).reshape(-1, 1)
    bias_total = bw_b2 + sg_b2 + tgt_b + dep_b + fan_b
    Wcat = jnp.concatenate([bw_W2, sg_W2, type_table, tgt_W, dep_W,
                            fan_W, bias_total[None, :]], axis=0)

    out2 = _dense_call(gathered, mk, w1cat, b1cat, Wcat, pos_tiled,
                       ln_g, ln_b, R=R)
    return out2.reshape(B, L, D)


# speedup vs baseline: 8.1472x; 1.0840x over previous
import functools

import jax
import jax.numpy as jnp
from jax import lax
from jax.experimental import pallas as pl
from jax.experimental.pallas import tpu as pltpu
from jax.experimental.pallas import tpu_sc as plsc


_NCHUNK = 4
_CHUNK = 80
_R = 3200
_H1 = 64
_H2 = 32
_NT = 6


def _make_sc_gather(V, D, CL):
    info = plsc.get_sparse_core_info()
    NC, NS = info.num_cores, info.num_subcores
    NW = NC * NS
    assert CL % (NW * _CHUNK) == 0
    per_w = CL // NW
    n_chunks = per_w // _CHUNK
    assert n_chunks % 2 == 0

    mesh = plsc.VectorSubcoreMesh(core_axis_name="c", subcore_axis_name="s")

    @functools.partial(
        pl.kernel,
        mesh=mesh,
        out_type=jax.ShapeDtypeStruct((CL, D), jnp.float32),
        scratch_types=[
            pltpu.VMEM((n_chunks, _CHUNK), jnp.int32),
            pltpu.VMEM((_CHUNK, D), jnp.float32),
            pltpu.VMEM((_CHUNK, D), jnp.float32),
            pltpu.SemaphoreType.DMA,
            pltpu.SemaphoreType.DMA,
        ],
    )
    def gather_kernel(table_hbm, ids_hbm, out_hbm, idx_v, buf_a, buf_b, sem_a, sem_b):
        wid = lax.axis_index("s") * NC + lax.axis_index("c")
        pltpu.sync_copy(ids_hbm.at[wid], idx_v)
        base = wid * per_w

        def gstart(j, buf, sem):
            pltpu.async_copy(table_hbm.at[idx_v.at[j]], buf, sem)

        def gwait(j, buf, sem):
            pltpu.make_async_copy(table_hbm.at[idx_v.at[j]], buf, sem).wait()

        def store(j, buf):
            pltpu.sync_copy(buf, out_hbm.at[pl.ds(base + j * _CHUNK, _CHUNK), :])

        gstart(0, buf_a, sem_a)

        def step(i, carry):
            j0 = 2 * i
            j1 = j0 + 1
            gwait(j0, buf_a, sem_a)
            gstart(j1, buf_b, sem_b)
            store(j0, buf_a)
            gwait(j1, buf_b, sem_b)

            @pl.when(i < n_chunks // 2 - 1)
            def _():
                gstart(j0 + 2, buf_a, sem_a)

            store(j1, buf_b)
            return carry

        lax.fori_loop(0, n_chunks // 2, step, 0)

    return gather_kernel


def _dense_body(g_ref, mk_ref, w1_ref, b1_ref, Wc_ref, pos_ref,
                lng_ref, lnb_ref, *rest):
    out_ref = rest[-1]
    x = g_ref[...]
    R, D = x.shape
    H = _H1 + _H2

    mk = mk_ref[...]
    bw = mk[0:1, :]
    sg = mk[1:2, :]
    rank1 = mk[2:5, :]
    phyf = mk[5:6, :]

    drive = jnp.where(lax.broadcasted_iota(jnp.int32, (H, 1), 0) < _H1, bw, sg)
    hid = jnp.maximum(w1_ref[...] * drive + b1_ref[...], 0.0)

    oh = (phyf == lax.broadcasted_iota(jnp.int32, (_NT, 1), 0).astype(jnp.float32)
          ).astype(jnp.float32)

    ones = jnp.full((1, R), 1.0, jnp.float32)
    F = jnp.concatenate([hid, oh, rank1, ones], axis=0)

    x = x + lax.dot_general(F, Wc_ref[...], (((0,), (0,)), ((), ())),
                            preferred_element_type=jnp.float32)
    x = x + pos_ref[...]

    onesD = jnp.full((D, 1), 1.0 / D, jnp.float32)
    mu = jnp.dot(x, onesD, preferred_element_type=jnp.float32)
    ms = jnp.dot(x * x, onesD, preferred_element_type=jnp.float32)
    var = ms - mu * mu
    s = lax.rsqrt(var + 1e-5) * lng_ref[...][None, :]
    out_ref[...] = (x - mu) * s + lnb_ref[...][None, :]


def _dense_call(g2, mk, w1cat, b1cat, Wcat, pos_tiled, lng, lnb,
                prev, blk0, BL):
    CL, D = g2.shape
    R = _R

    def rep2(i):
        return (0, 0)

    def rep1(i):
        return (0,)

    vecD = pl.BlockSpec(lnb.shape, rep1)
    in_specs = [
        pl.BlockSpec((R, D), lambda i: (i, 0)),
        pl.BlockSpec((mk.shape[0], R), lambda i, b=blk0: (0, i + b)),
        pl.BlockSpec(w1cat.shape, rep2), pl.BlockSpec(b1cat.shape, rep2),
        pl.BlockSpec(Wcat.shape, rep2),
        pl.BlockSpec((R, D), rep2),
        vecD, vecD,
    ]
    args = [g2, mk, w1cat, b1cat, Wcat, pos_tiled, lng, lnb]
    io_aliases = {}
    if prev is not None:
        in_specs.append(pl.BlockSpec(memory_space=pltpu.MemorySpace.HBM))
        args.append(prev)
        io_aliases = {len(args) - 1: 0}

    return pl.pallas_call(
        _dense_body,
        grid=(CL // R,),
        in_specs=in_specs,
        out_specs=pl.BlockSpec((R, D), lambda i, b=blk0: (i + b, 0)),
        out_shape=jax.ShapeDtypeStruct((BL, D), jnp.float32),
        input_output_aliases=io_aliases,
        compiler_params=pltpu.CompilerParams(
            dimension_semantics=("arbitrary",)),
    )(*args)


def kernel(token_ids, bitwidths, signed_flags, phy_types, target_mask,
           dependency_mask, fanout_mask, token_table, bw_W1, bw_b1, bw_W2,
           bw_b2, sg_W1, sg_b1, sg_W2, sg_b2, type_table, tgt_W, tgt_b,
           dep_W, dep_b, fan_W, fan_b, pos_table, ln_g, ln_b):
    B, L = token_ids.shape
    V, D = token_table.shape
    BL = B * L
    CL = BL // _NCHUNK

    info = plsc.get_sparse_core_info()
    nw = info.num_cores * info.num_subcores
    ids4 = token_ids.astype(jnp.int32).reshape(
        _NCHUNK, nw, CL // (nw * _CHUNK), _CHUNK)

    assert _R % L == 0 and CL % _R == 0 and _R % 128 == 0 and CL % L == 0
    pos_tiled = jnp.tile(pos_table[:L], (_R // L, 1))

    def row(a):
        return a.reshape(1, BL).astype(jnp.float32)

    mk = jnp.concatenate([row(bitwidths), row(signed_flags), row(target_mask),
                          row(dependency_mask), row(fanout_mask),
                          row(phy_types)], axis=0)

    w1cat = jnp.concatenate([bw_W1.reshape(-1), sg_W1.reshape(-1)]).reshape(-1, 1)
    b1cat = jnp.concatenate([bw_b1, sg_b1]).reshape(-1, 1)
    bias_total = bw_b2 + sg_b2 + tgt_b + dep_b + fan_b
    Wcat = jnp.concatenate([bw_W2, sg_W2, type_table, tgt_W, dep_W,
                            fan_W, bias_total[None, :]], axis=0)

    sc_gather = _make_sc_gather(V, D, CL)
    gs = [sc_gather(token_table, ids4[c]) for c in range(_NCHUNK)]

    buf = None
    for c in range(_NCHUNK):
        buf = _dense_call(gs[c], mk, w1cat, b1cat, Wcat, pos_tiled,
                          ln_g, ln_b, prev=buf, blk0=c * (CL // _R), BL=BL)
    return buf.reshape(B, L, D)


# speedup vs baseline: 8.4506x; 1.0372x over previous
import functools

import jax
import jax.numpy as jnp
from jax import lax
from jax.experimental import pallas as pl
from jax.experimental.pallas import tpu as pltpu
from jax.experimental.pallas import tpu_sc as plsc


_NCHUNK = 2
_CHUNK = 128
_R = 3200
_H1 = 64
_H2 = 32
_NT = 6


def _make_sc_gather(V, D, CL):
    info = plsc.get_sparse_core_info()
    NC, NS = info.num_cores, info.num_subcores
    NW = NC * NS
    assert CL % (NW * _CHUNK) == 0
    per_w = CL // NW
    n_chunks = per_w // _CHUNK

    mesh = plsc.VectorSubcoreMesh(core_axis_name="c", subcore_axis_name="s")

    @functools.partial(
        pl.kernel,
        mesh=mesh,
        out_type=jax.ShapeDtypeStruct((CL, D), jnp.float32),
        scratch_types=[
            pltpu.VMEM((n_chunks, _CHUNK), jnp.int32),
            pltpu.VMEM((_CHUNK, D), jnp.float32),
            pltpu.VMEM((_CHUNK, D), jnp.float32),
            pltpu.SemaphoreType.DMA,
            pltpu.SemaphoreType.DMA,
        ],
    )
    def gather_kernel(table_hbm, ids_hbm, out_hbm, idx_v, buf_a, buf_b, sem_a, sem_b):
        wid = lax.axis_index("s") * NC + lax.axis_index("c")
        pltpu.sync_copy(ids_hbm.at[wid], idx_v)
        base = wid * per_w

        def gstart(j, buf, sem):
            pltpu.async_copy(table_hbm.at[idx_v.at[j]], buf, sem)

        def gwait(j, buf, sem):
            pltpu.make_async_copy(table_hbm.at[idx_v.at[j]], buf, sem).wait()

        def store(j, buf):
            pltpu.sync_copy(buf, out_hbm.at[pl.ds(base + j * _CHUNK, _CHUNK), :])

        gstart(0, buf_a, sem_a)

        def step(i, carry):
            j0 = 2 * i
            j1 = j0 + 1
            gwait(j0, buf_a, sem_a)
            gstart(j1, buf_b, sem_b)
            store(j0, buf_a)
            gwait(j1, buf_b, sem_b)

            @pl.when(j0 + 2 < n_chunks)
            def _():
                gstart(j0 + 2, buf_a, sem_a)

            store(j1, buf_b)
            return carry

        lax.fori_loop(0, n_chunks // 2, step, 0)

        if n_chunks % 2:
            last = n_chunks - 1
            gwait(last, buf_a, sem_a)
            store(last, buf_a)

    return gather_kernel


def _dense_body(g_ref, mk_ref, w1_ref, b1_ref, Wc_ref, pos_ref,
                lng_ref, lnb_ref, *rest):
    out_ref = rest[-1]
    x = g_ref[...]
    R, D = x.shape
    H = _H1 + _H2

    mk = mk_ref[...]
    bw = mk[0:1, :]
    sg = mk[1:2, :]
    rank1 = mk[2:5, :]
    phyf = mk[5:6, :]

    drive = jnp.where(lax.broadcasted_iota(jnp.int32, (H, 1), 0) < _H1, bw, sg)
    hid = jnp.maximum(w1_ref[...] * drive + b1_ref[...], 0.0)

    oh = (phyf == lax.broadcasted_iota(jnp.int32, (_NT, 1), 0).astype(jnp.float32)
          ).astype(jnp.float32)

    ones = jnp.full((1, R), 1.0, jnp.float32)
    F = jnp.concatenate([hid, oh, rank1, ones], axis=0)

    x = x + lax.dot_general(F, Wc_ref[...], (((0,), (0,)), ((), ())),
                            preferred_element_type=jnp.float32)
    x = x + pos_ref[...]

    onesD = jnp.full((D, 1), 1.0 / D, jnp.float32)
    mu = jnp.dot(x, onesD, preferred_element_type=jnp.float32)
    ms = jnp.dot(x * x, onesD, preferred_element_type=jnp.float32)
    var = ms - mu * mu
    s = lax.rsqrt(var + 1e-5) * lng_ref[...][None, :]
    out_ref[...] = (x - mu) * s + lnb_ref[...][None, :]


def _dense_call(g2, mk, w1cat, b1cat, Wcat, pos_tiled, lng, lnb,
                prev, blk0, BL):
    CL, D = g2.shape
    R = _R

    def rep2(i):
        return (0, 0)

    def rep1(i):
        return (0,)

    vecD = pl.BlockSpec(lnb.shape, rep1)
    in_specs = [
        pl.BlockSpec((R, D), lambda i: (i, 0)),
        pl.BlockSpec((mk.shape[0], R), lambda i, b=blk0: (0, i + b)),
        pl.BlockSpec(w1cat.shape, rep2), pl.BlockSpec(b1cat.shape, rep2),
        pl.BlockSpec(Wcat.shape, rep2),
        pl.BlockSpec((R, D), rep2),
        vecD, vecD,
    ]
    args = [g2, mk, w1cat, b1cat, Wcat, pos_tiled, lng, lnb]
    io_aliases = {}
    if prev is not None:
        in_specs.append(pl.BlockSpec(memory_space=pltpu.MemorySpace.HBM))
        args.append(prev)
        io_aliases = {len(args) - 1: 0}

    return pl.pallas_call(
        _dense_body,
        grid=(CL // R,),
        in_specs=in_specs,
        out_specs=pl.BlockSpec((R, D), lambda i, b=blk0: (i + b, 0)),
        out_shape=jax.ShapeDtypeStruct((BL, D), jnp.float32),
        input_output_aliases=io_aliases,
        compiler_params=pltpu.CompilerParams(
            dimension_semantics=("arbitrary",)),
    )(*args)


def kernel(token_ids, bitwidths, signed_flags, phy_types, target_mask,
           dependency_mask, fanout_mask, token_table, bw_W1, bw_b1, bw_W2,
           bw_b2, sg_W1, sg_b1, sg_W2, sg_b2, type_table, tgt_W, tgt_b,
           dep_W, dep_b, fan_W, fan_b, pos_table, ln_g, ln_b):
    B, L = token_ids.shape
    V, D = token_table.shape
    BL = B * L
    CL = BL // _NCHUNK

    info = plsc.get_sparse_core_info()
    nw = info.num_cores * info.num_subcores
    ids4 = token_ids.astype(jnp.int32).reshape(
        _NCHUNK, nw, CL // (nw * _CHUNK), _CHUNK)

    assert _R % L == 0 and CL % _R == 0 and _R % 128 == 0 and CL % L == 0
    pos_tiled = jnp.tile(pos_table[:L], (_R // L, 1))

    def row(a):
        return a.reshape(1, BL).astype(jnp.float32)

    mk = jnp.concatenate([row(bitwidths), row(signed_flags), row(target_mask),
                          row(dependency_mask), row(fanout_mask),
                          row(phy_types)], axis=0)

    w1cat = jnp.concatenate([bw_W1.reshape(-1), sg_W1.reshape(-1)]).reshape(-1, 1)
    b1cat = jnp.concatenate([bw_b1, sg_b1]).reshape(-1, 1)
    bias_total = bw_b2 + sg_b2 + tgt_b + dep_b + fan_b
    Wcat = jnp.concatenate([bw_W2, sg_W2, type_table, tgt_W, dep_W,
                            fan_W, bias_total[None, :]], axis=0)

    sc_gather = _make_sc_gather(V, D, CL)
    gs = [sc_gather(token_table, ids4[c]) for c in range(_NCHUNK)]

    buf = None
    for c in range(_NCHUNK):
        buf = _dense_call(gs[c], mk, w1cat, b1cat, Wcat, pos_tiled,
                          ln_g, ln_b, prev=buf, blk0=c * (CL // _R), BL=BL)
    return buf.reshape(B, L, D)


# speedup vs baseline: 9.0577x; 1.0718x over previous
import functools

import jax
import jax.numpy as jnp
from jax import lax
from jax.experimental import pallas as pl
from jax.experimental.pallas import tpu as pltpu
from jax.experimental.pallas import tpu_sc as plsc


_NCHUNK = 2
_CHUNK = 128
_R = 6400
_H1 = 64
_H2 = 32
_NT = 6


def _make_sc_gather(V, D, CL):
    info = plsc.get_sparse_core_info()
    NC, NS = info.num_cores, info.num_subcores
    NW = NC * NS
    assert CL % (NW * _CHUNK) == 0
    per_w = CL // NW
    n_chunks = per_w // _CHUNK

    mesh = plsc.VectorSubcoreMesh(core_axis_name="c", subcore_axis_name="s")

    @functools.partial(
        pl.kernel,
        mesh=mesh,
        out_type=jax.ShapeDtypeStruct((CL, D), jnp.float32),
        scratch_types=[
            pltpu.VMEM((n_chunks, _CHUNK), jnp.int32),
            pltpu.VMEM((_CHUNK, D), jnp.float32),
            pltpu.VMEM((_CHUNK, D), jnp.float32),
            pltpu.SemaphoreType.DMA,
            pltpu.SemaphoreType.DMA,
        ],
    )
    def gather_kernel(table_hbm, ids_hbm, out_hbm, idx_v, buf_a, buf_b, sem_a, sem_b):
        wid = lax.axis_index("s") * NC + lax.axis_index("c")
        pltpu.sync_copy(ids_hbm.at[wid], idx_v)
        base = wid * per_w

        def gstart(j, buf, sem):
            pltpu.async_copy(table_hbm.at[idx_v.at[j]], buf, sem)

        def gwait(j, buf, sem):
            pltpu.make_async_copy(table_hbm.at[idx_v.at[j]], buf, sem).wait()

        def store(j, buf):
            pltpu.sync_copy(buf, out_hbm.at[pl.ds(base + j * _CHUNK, _CHUNK), :])

        gstart(0, buf_a, sem_a)

        def step(i, carry):
            j0 = 2 * i
            j1 = j0 + 1
            gwait(j0, buf_a, sem_a)
            gstart(j1, buf_b, sem_b)
            store(j0, buf_a)
            gwait(j1, buf_b, sem_b)

            @pl.when(j0 + 2 < n_chunks)
            def _():
                gstart(j0 + 2, buf_a, sem_a)

            store(j1, buf_b)
            return carry

        lax.fori_loop(0, n_chunks // 2, step, 0)

        if n_chunks % 2:
            last = n_chunks - 1
            gwait(last, buf_a, sem_a)
            store(last, buf_a)

    return gather_kernel


def _dense_body(g_ref, mk_ref, w1_ref, b1_ref, Wc_ref, pos_ref,
                lng_ref, lnb_ref, *rest):
    out_ref = rest[-1]
    x = g_ref[...]
    R, D = x.shape
    H = _H1 + _H2

    mk = mk_ref[...]
    bw = mk[0:1, :]
    sg = mk[1:2, :]
    rank1 = mk[2:5, :]
    phyf = mk[5:6, :]

    drive = jnp.where(lax.broadcasted_iota(jnp.int32, (H, 1), 0) < _H1, bw, sg)
    hid = jnp.maximum(w1_ref[...] * drive + b1_ref[...], 0.0)

    oh = (phyf == lax.broadcasted_iota(jnp.int32, (_NT, 1), 0).astype(jnp.float32)
          ).astype(jnp.float32)

    ones = jnp.full((1, R), 1.0, jnp.float32)
    F = jnp.concatenate([hid, oh, rank1, ones], axis=0)

    x = x + lax.dot_general(F, Wc_ref[...], (((0,), (0,)), ((), ())),
                            preferred_element_type=jnp.float32)
    x = x + pos_ref[...]

    onesD = jnp.full((D, 1), 1.0 / D, jnp.float32)
    mu = jnp.dot(x, onesD, preferred_element_type=jnp.float32)
    ms = jnp.dot(x * x, onesD, preferred_element_type=jnp.float32)
    var = ms - mu * mu
    s = lax.rsqrt(var + 1e-5) * lng_ref[...][None, :]
    out_ref[...] = (x - mu) * s + lnb_ref[...][None, :]


def _dense_call(g2, mk, w1cat, b1cat, Wcat, pos_tiled, lng, lnb,
                prev, blk0, BL):
    CL, D = g2.shape
    R = _R

    def rep2(i):
        return (0, 0)

    def rep1(i):
        return (0,)

    vecD = pl.BlockSpec(lnb.shape, rep1)
    in_specs = [
        pl.BlockSpec((R, D), lambda i: (i, 0)),
        pl.BlockSpec((mk.shape[0], R), lambda i, b=blk0: (0, i + b)),
        pl.BlockSpec(w1cat.shape, rep2), pl.BlockSpec(b1cat.shape, rep2),
        pl.BlockSpec(Wcat.shape, rep2),
        pl.BlockSpec((R, D), rep2),
        vecD, vecD,
    ]
    args = [g2, mk, w1cat, b1cat, Wcat, pos_tiled, lng, lnb]
    io_aliases = {}
    if prev is not None:
        in_specs.append(pl.BlockSpec(memory_space=pltpu.MemorySpace.HBM))
        args.append(prev)
        io_aliases = {len(args) - 1: 0}

    return pl.pallas_call(
        _dense_body,
        grid=(CL // R,),
        in_specs=in_specs,
        out_specs=pl.BlockSpec((R, D), lambda i, b=blk0: (i + b, 0)),
        out_shape=jax.ShapeDtypeStruct((BL, D), jnp.float32),
        input_output_aliases=io_aliases,
        compiler_params=pltpu.CompilerParams(
            dimension_semantics=("arbitrary",)),
    )(*args)


def kernel(token_ids, bitwidths, signed_flags, phy_types, target_mask,
           dependency_mask, fanout_mask, token_table, bw_W1, bw_b1, bw_W2,
           bw_b2, sg_W1, sg_b1, sg_W2, sg_b2, type_table, tgt_W, tgt_b,
           dep_W, dep_b, fan_W, fan_b, pos_table, ln_g, ln_b):
    B, L = token_ids.shape
    V, D = token_table.shape
    BL = B * L
    CL = BL // _NCHUNK

    info = plsc.get_sparse_core_info()
    nw = info.num_cores * info.num_subcores
    ids4 = token_ids.astype(jnp.int32).reshape(
        _NCHUNK, nw, CL // (nw * _CHUNK), _CHUNK)

    assert _R % L == 0 and CL % _R == 0 and _R % 128 == 0 and CL % L == 0
    pos_tiled = jnp.tile(pos_table[:L], (_R // L, 1))

    def row(a):
        return a.reshape(1, BL).astype(jnp.float32)

    mk = jnp.concatenate([row(bitwidths), row(signed_flags), row(target_mask),
                          row(dependency_mask), row(fanout_mask),
                          row(phy_types)], axis=0)

    w1cat = jnp.concatenate([bw_W1.reshape(-1), sg_W1.reshape(-1)]).reshape(-1, 1)
    b1cat = jnp.concatenate([bw_b1, sg_b1]).reshape(-1, 1)
    bias_total = bw_b2 + sg_b2 + tgt_b + dep_b + fan_b
    Wcat = jnp.concatenate([bw_W2, sg_W2, type_table, tgt_W, dep_W,
                            fan_W, bias_total[None, :]], axis=0)

    sc_gather = _make_sc_gather(V, D, CL)
    gs = [sc_gather(token_table, ids4[c]) for c in range(_NCHUNK)]

    buf = None
    for c in range(_NCHUNK):
        buf = _dense_call(gs[c], mk, w1cat, b1cat, Wcat, pos_tiled,
                          ln_g, ln_b, prev=buf, blk0=c * (CL // _R), BL=BL)
    return buf.reshape(B, L, D)


# speedup vs baseline: 9.0990x; 1.0046x over previous
import functools

import jax
import jax.numpy as jnp
from jax import lax
from jax.experimental import pallas as pl
from jax.experimental.pallas import tpu as pltpu
from jax.experimental.pallas import tpu_sc as plsc


_NCHUNK = 2
_CHUNK = 128
_R = 12800
_H1 = 64
_H2 = 32
_NT = 6


def _make_sc_gather(V, D, CL):
    info = plsc.get_sparse_core_info()
    NC, NS = info.num_cores, info.num_subcores
    NW = NC * NS
    assert CL % (NW * _CHUNK) == 0
    per_w = CL // NW
    n_chunks = per_w // _CHUNK

    mesh = plsc.VectorSubcoreMesh(core_axis_name="c", subcore_axis_name="s")

    @functools.partial(
        pl.kernel,
        mesh=mesh,
        out_type=jax.ShapeDtypeStruct((CL, D), jnp.float32),
        scratch_types=[
            pltpu.VMEM((n_chunks, _CHUNK), jnp.int32),
            pltpu.VMEM((_CHUNK, D), jnp.float32),
            pltpu.VMEM((_CHUNK, D), jnp.float32),
            pltpu.SemaphoreType.DMA,
            pltpu.SemaphoreType.DMA,
        ],
    )
    def gather_kernel(table_hbm, ids_hbm, out_hbm, idx_v, buf_a, buf_b, sem_a, sem_b):
        wid = lax.axis_index("s") * NC + lax.axis_index("c")
        pltpu.sync_copy(ids_hbm.at[wid], idx_v)
        base = wid * per_w

        def gstart(j, buf, sem):
            pltpu.async_copy(table_hbm.at[idx_v.at[j]], buf, sem)

        def gwait(j, buf, sem):
            pltpu.make_async_copy(table_hbm.at[idx_v.at[j]], buf, sem).wait()

        def store(j, buf):
            pltpu.sync_copy(buf, out_hbm.at[pl.ds(base + j * _CHUNK, _CHUNK), :])

        gstart(0, buf_a, sem_a)

        def step(i, carry):
            j0 = 2 * i
            j1 = j0 + 1
            gwait(j0, buf_a, sem_a)
            gstart(j1, buf_b, sem_b)
            store(j0, buf_a)
            gwait(j1, buf_b, sem_b)

            @pl.when(j0 + 2 < n_chunks)
            def _():
                gstart(j0 + 2, buf_a, sem_a)

            store(j1, buf_b)
            return carry

        lax.fori_loop(0, n_chunks // 2, step, 0)

        if n_chunks % 2:
            last = n_chunks - 1
            gwait(last, buf_a, sem_a)
            store(last, buf_a)

    return gather_kernel


def _dense_body(g_ref, mk_ref, w1_ref, b1_ref, Wc_ref, pos_ref,
                lng_ref, lnb_ref, *rest):
    out_ref = rest[-1]
    x = g_ref[...]
    R, D = x.shape
    H = _H1 + _H2

    mk = mk_ref[...]
    bw = mk[0:1, :]
    sg = mk[1:2, :]
    rank1 = mk[2:5, :]
    phyf = mk[5:6, :]

    drive = jnp.where(lax.broadcasted_iota(jnp.int32, (H, 1), 0) < _H1, bw, sg)
    hid = jnp.maximum(w1_ref[...] * drive + b1_ref[...], 0.0)

    oh = (phyf == lax.broadcasted_iota(jnp.int32, (_NT, 1), 0).astype(jnp.float32)
          ).astype(jnp.float32)

    ones = jnp.full((1, R), 1.0, jnp.float32)
    F = jnp.concatenate([hid, oh, rank1, ones], axis=0)

    x = x + lax.dot_general(F, Wc_ref[...], (((0,), (0,)), ((), ())),
                            preferred_element_type=jnp.float32)
    x = x + pos_ref[...]

    onesD = jnp.full((D, 1), 1.0 / D, jnp.float32)
    mu = jnp.dot(x, onesD, preferred_element_type=jnp.float32)
    ms = jnp.dot(x * x, onesD, preferred_element_type=jnp.float32)
    var = ms - mu * mu
    s = lax.rsqrt(var + 1e-5) * lng_ref[...][None, :]
    out_ref[...] = (x - mu) * s + lnb_ref[...][None, :]


def _dense_call(g2, mk, w1cat, b1cat, Wcat, pos_tiled, lng, lnb,
                prev, blk0, BL):
    CL, D = g2.shape
    R = _R

    def rep2(i):
        return (0, 0)

    def rep1(i):
        return (0,)

    vecD = pl.BlockSpec(lnb.shape, rep1)
    in_specs = [
        pl.BlockSpec((R, D), lambda i: (i, 0)),
        pl.BlockSpec((mk.shape[0], R), lambda i, b=blk0: (0, i + b)),
        pl.BlockSpec(w1cat.shape, rep2), pl.BlockSpec(b1cat.shape, rep2),
        pl.BlockSpec(Wcat.shape, rep2),
        pl.BlockSpec((R, D), rep2),
        vecD, vecD,
    ]
    args = [g2, mk, w1cat, b1cat, Wcat, pos_tiled, lng, lnb]
    io_aliases = {}
    if prev is not None:
        in_specs.append(pl.BlockSpec(memory_space=pltpu.MemorySpace.HBM))
        args.append(prev)
        io_aliases = {len(args) - 1: 0}

    return pl.pallas_call(
        _dense_body,
        grid=(CL // R,),
        in_specs=in_specs,
        out_specs=pl.BlockSpec((R, D), lambda i, b=blk0: (i + b, 0)),
        out_shape=jax.ShapeDtypeStruct((BL, D), jnp.float32),
        input_output_aliases=io_aliases,
        compiler_params=pltpu.CompilerParams(
            dimension_semantics=("arbitrary",)),
    )(*args)


def kernel(token_ids, bitwidths, signed_flags, phy_types, target_mask,
           dependency_mask, fanout_mask, token_table, bw_W1, bw_b1, bw_W2,
           bw_b2, sg_W1, sg_b1, sg_W2, sg_b2, type_table, tgt_W, tgt_b,
           dep_W, dep_b, fan_W, fan_b, pos_table, ln_g, ln_b):
    B, L = token_ids.shape
    V, D = token_table.shape
    BL = B * L
    CL = BL // _NCHUNK

    info = plsc.get_sparse_core_info()
    nw = info.num_cores * info.num_subcores
    ids4 = token_ids.astype(jnp.int32).reshape(
        _NCHUNK, nw, CL // (nw * _CHUNK), _CHUNK)

    assert _R % L == 0 and CL % _R == 0 and _R % 128 == 0 and CL % L == 0
    pos_tiled = jnp.tile(pos_table[:L], (_R // L, 1))

    def row(a):
        return a.reshape(1, BL).astype(jnp.float32)

    mk = jnp.concatenate([row(bitwidths), row(signed_flags), row(target_mask),
                          row(dependency_mask), row(fanout_mask),
                          row(phy_types)], axis=0)

    w1cat = jnp.concatenate([bw_W1.reshape(-1), sg_W1.reshape(-1)]).reshape(-1, 1)
    b1cat = jnp.concatenate([bw_b1, sg_b1]).reshape(-1, 1)
    bias_total = bw_b2 + sg_b2 + tgt_b + dep_b + fan_b
    Wcat = jnp.concatenate([bw_W2, sg_W2, type_table, tgt_W, dep_W,
                            fan_W, bias_total[None, :]], axis=0)

    sc_gather = _make_sc_gather(V, D, CL)
    gs = [sc_gather(token_table, ids4[c]) for c in range(_NCHUNK)]

    buf = None
    for c in range(_NCHUNK):
        buf = _dense_call(gs[c], mk, w1cat, b1cat, Wcat, pos_tiled,
                          ln_g, ln_b, prev=buf, blk0=c * (CL // _R), BL=BL)
    return buf.reshape(B, L, D)
